# baseline (speedup 1.0000x reference)
import functools

import jax
import jax.numpy as jnp
from jax import lax
from jax.experimental import pallas as pl
from jax.experimental.pallas import tpu as pltpu
from jax.experimental.pallas import tpu_sc as plsc

NN = 10000
NE = 320000
D = 128
DE = 16
NL = 4

NC, NS, L = 2, 16, 16
NW = NC * NS
NEP = 321536
EPW = NEP // NW
EB = 64
NBLK = EPW // EB
NNP = 10240
NPT = NNP // NS


def _edge_body(hw_hbm, e_hbm, src_hbm, dst_hbm, zeros_hbm, out_hbm,
               src_v, dst_v, rows_v, e_v, agg_sh, gsem, esem):
    cid = lax.axis_index("c")
    sid = lax.axis_index("s")
    pltpu.sync_copy(zeros_hbm.at[pl.ds(sid * NPT, NPT)],
                    agg_sh.at[pl.ds(sid * NPT, NPT)])
    plsc.subcore_barrier()

    base = (cid * NS + sid) * EPW

    def blk(b, carry):
        off = base + b * EB
        pltpu.sync_copy(src_hbm.at[pl.ds(off, EB)], src_v)
        pltpu.sync_copy(dst_hbm.at[pl.ds(off, EB)], dst_v)
        g = pltpu.make_async_copy(hw_hbm.at[src_v], rows_v, gsem)
        e = pltpu.make_async_copy(e_hbm.at[pl.ds(off, EB)], e_v, esem)
        g.start()
        e.start()
        g.wait()
        e.wait()

        def row(r, c2):
            for k in range(D // L):
                sl = pl.ds(k * L, L)
                rows_v[r, sl] = jnp.maximum(rows_v[r, sl] + e_v[r, sl], 0.0)
            return c2

        lax.fori_loop(0, EB, row, 0)
        pltpu.sync_copy(rows_v, agg_sh.at[dst_v], add=True)
        return carry

    lax.fori_loop(0, NBLK, blk, 0)
    plsc.subcore_barrier()
    pltpu.sync_copy(agg_sh.at[pl.ds(sid * NPT, NPT)],
                    out_hbm.at[cid, pl.ds(sid * NPT, NPT)])


_EDGE_KERNEL = []


def _edge_kernel():
    if not _EDGE_KERNEL:
        mesh = plsc.VectorSubcoreMesh(core_axis_name="c", subcore_axis_name="s",
                                      num_cores=NC, num_subcores=NS)
        _EDGE_KERNEL.append(pl.kernel(
            _edge_body,
            out_type=jax.ShapeDtypeStruct((NC, NNP, D), jnp.float32),
            mesh=mesh,
            scratch_types=[
                pltpu.VMEM((EB,), jnp.int32),
                pltpu.VMEM((EB,), jnp.int32),
                pltpu.VMEM((EB, D), jnp.float32),
                pltpu.VMEM((EB, D), jnp.float32),
                pltpu.VMEM_SHARED((NNP, D), jnp.float32),
                pltpu.SemaphoreType.DMA,
                pltpu.SemaphoreType.DMA,
            ],
        ))
    return _EDGE_KERNEL[0]


def _mm(a, w):
    M, K = a.shape
    N = w.shape[1]
    BM = 2000

    def body(a_ref, w_ref, o_ref):
        o_ref[...] = jnp.dot(a_ref[...], w_ref[...],
                             preferred_element_type=jnp.float32)

    return pl.pallas_call(
        body,
        out_shape=jax.ShapeDtypeStruct((M, N), jnp.float32),
        grid=(M // BM,),
        in_specs=[pl.BlockSpec((BM, K), lambda i: (i, 0)),
                  pl.BlockSpec((K, N), lambda i: (0, 0))],
        out_specs=pl.BlockSpec((BM, N), lambda i: (i, 0)),
    )(a, w)


def _e_pre(A, Wc):
    K = A.shape[1]
    BE_ = 4096
    NB = NEP // BE_ + 1

    def body(a_ref, w_ref, o_ref):
        o_ref[...] = jnp.dot(a_ref[...], w_ref[...],
                             preferred_element_type=jnp.float32)

    return pl.pallas_call(
        body,
        out_shape=jax.ShapeDtypeStruct((NEP, D), jnp.float32),
        grid=(NB,),
        in_specs=[pl.BlockSpec((BE_, K), lambda i: (i, 0)),
                  pl.BlockSpec((K, D), lambda i: (0, 0))],
        out_specs=pl.BlockSpec((BE_, D), lambda i: (i, 0)),
    )(A, Wc)


def _update(h, parts, Wu1, Wu2, b, Wxn):
    BM = 2000

    def body(h_ref, p_ref, w1_ref, w2_ref, b_ref, wx_ref, hn_ref, hw_ref):
        agg = p_ref[0] + p_ref[1]
        u = jnp.dot(h_ref[...], w1_ref[...], preferred_element_type=jnp.float32)
        u = u + jnp.dot(agg, w2_ref[...], preferred_element_type=jnp.float32)
        u = u + b_ref[...]
        hn = h_ref[...] + jnp.maximum(u, 0.0)
        hn_ref[...] = hn
        hw_ref[...] = jnp.dot(hn, wx_ref[...], preferred_element_type=jnp.float32)

    return pl.pallas_call(
        body,
        out_shape=(jax.ShapeDtypeStruct((NN, D), jnp.float32),
                   jax.ShapeDtypeStruct((NN, D), jnp.float32)),
        grid=(NN // BM,),
        in_specs=[pl.BlockSpec((BM, D), lambda i: (i, 0)),
                  pl.BlockSpec((NC, BM, D), lambda i: (0, i, 0)),
                  pl.BlockSpec((D, D), lambda i: (0, 0)),
                  pl.BlockSpec((D, D), lambda i: (0, 0)),
                  pl.BlockSpec((1, D), lambda i: (0, 0)),
                  pl.BlockSpec((D, D), lambda i: (0, 0))],
        out_specs=(pl.BlockSpec((BM, D), lambda i: (i, 0)),
                   pl.BlockSpec((BM, D), lambda i: (i, 0))),
    )(h, parts, Wu1, Wu2, b, Wxn)


def kernel(x, edge_index, batch, edge_attr, length, W_msg, b_msg, W_upd, b_upd):
    del batch
    pad = NEP - NE
    src = jnp.concatenate([edge_index[0], jnp.zeros((pad,), jnp.int32)])
    dst = jnp.concatenate([edge_index[1], jnp.full((pad,), NN, jnp.int32)])

    Wx = W_msg[:, :D, :]
    Wc = jnp.concatenate(
        [W_msg[:, D:D + DE + 1, :], b_msg[:, None, :]], axis=1)
    A = jnp.concatenate(
        [edge_attr, length[:, None], jnp.ones((NE, 1), jnp.float32)], axis=1)
    A = jnp.concatenate([A, jnp.zeros((pad, DE + 2), jnp.float32)], axis=0)

    zeros = jnp.zeros((NNP, D), jnp.float32)

    h = x
    hW = _mm(x, Wx[0])
    outs = []
    for i in range(NL):
        Ei = _e_pre(A, Wc[i])
        parts = _edge_kernel()(hW, Ei, src, dst, zeros)
        h, hW = _update(h, parts, W_upd[i, :D, :], W_upd[i, D:, :],
                        b_upd[i][None], Wx[(i + 1) % NL])
        if (i + 1) % 2 == 0:
            outs.append(h)
    return jnp.stack(outs)


# speedup vs baseline: 1.1723x; 1.1723x over previous
import functools

import jax
import jax.numpy as jnp
from jax import lax
from jax.experimental import pallas as pl
from jax.experimental.pallas import tpu as pltpu
from jax.experimental.pallas import tpu_sc as plsc

NN = 10000
NE = 320000
D = 128
DE = 16
NL = 4

NC, NS, L = 2, 16, 16
NW = NC * NS
EPW = NE // NW
EB = 80
NBLK = EPW // EB
NNP = 10240
NPT = NNP // NS


def _edge_body(hw_hbm, e_hbm, src_hbm, dst_hbm, zeros_hbm, out_hbm,
               src_v, dst_v, rows_v, e_v, agg_sh, gsem, esem):
    cid = lax.axis_index("c")
    sid = lax.axis_index("s")
    pltpu.sync_copy(zeros_hbm.at[pl.ds(sid * NPT, NPT)],
                    agg_sh.at[pl.ds(sid * NPT, NPT)])
    plsc.subcore_barrier()

    base = (cid * NS + sid) * EPW

    def blk(b, carry):
        off = base + b * EB
        pltpu.sync_copy(src_hbm.at[pl.ds(off, EB)], src_v)
        pltpu.sync_copy(dst_hbm.at[pl.ds(off, EB)], dst_v)
        g = pltpu.make_async_copy(hw_hbm.at[src_v], rows_v, gsem)
        e = pltpu.make_async_copy(e_hbm.at[pl.ds(off, EB)], e_v, esem)
        g.start()
        e.start()
        g.wait()
        e.wait()

        def row(r, c2):
            for k in range(D // L):
                sl = pl.ds(k * L, L)
                rows_v[r, sl] = jnp.maximum(rows_v[r, sl] + e_v[r, sl], 0.0)
            return c2

        lax.fori_loop(0, EB, row, 0)
        pltpu.sync_copy(rows_v, agg_sh.at[dst_v], add=True)
        return carry

    lax.fori_loop(0, NBLK, blk, 0)
    plsc.subcore_barrier()
    pltpu.sync_copy(agg_sh.at[pl.ds(sid * NPT, NPT)],
                    out_hbm.at[cid, pl.ds(sid * NPT, NPT)])


_EDGE_KERNEL = []


def _edge_kernel():
    if not _EDGE_KERNEL:
        mesh = plsc.VectorSubcoreMesh(core_axis_name="c", subcore_axis_name="s",
                                      num_cores=NC, num_subcores=NS)
        _EDGE_KERNEL.append(pl.kernel(
            _edge_body,
            out_type=jax.ShapeDtypeStruct((NC, NNP, D), jnp.float32),
            mesh=mesh,
            scratch_types=[
                pltpu.VMEM((EB,), jnp.int32),
                pltpu.VMEM((EB,), jnp.int32),
                pltpu.VMEM((EB, D), jnp.float32),
                pltpu.VMEM((EB, D), jnp.float32),
                pltpu.VMEM_SHARED((NNP, D), jnp.float32),
                pltpu.SemaphoreType.DMA,
                pltpu.SemaphoreType.DMA,
            ],
        ))
    return _EDGE_KERNEL[0]


def _mm(a, w):
    M, K = a.shape
    N = w.shape[1]
    BM = 2000

    def body(a_ref, w_ref, o_ref):
        o_ref[...] = jnp.dot(a_ref[...], w_ref[...],
                             preferred_element_type=jnp.float32)

    return pl.pallas_call(
        body,
        out_shape=jax.ShapeDtypeStruct((M, N), jnp.float32),
        grid=(M // BM,),
        in_specs=[pl.BlockSpec((BM, K), lambda i: (i, 0)),
                  pl.BlockSpec((K, N), lambda i: (0, 0))],
        out_specs=pl.BlockSpec((BM, N), lambda i: (i, 0)),
    )(a, w)


def _e_pre(A, Wc):
    K = A.shape[1]
    BE_ = 4000
    NB = NE // BE_

    def body(a_ref, w_ref, o_ref):
        o_ref[...] = jnp.dot(a_ref[...], w_ref[...],
                             preferred_element_type=jnp.float32)

    return pl.pallas_call(
        body,
        out_shape=jax.ShapeDtypeStruct((NE, D), jnp.float32),
        grid=(NB,),
        in_specs=[pl.BlockSpec((BE_, K), lambda i: (i, 0)),
                  pl.BlockSpec((K, D), lambda i: (0, 0))],
        out_specs=pl.BlockSpec((BE_, D), lambda i: (i, 0)),
    )(A, Wc)


def _update(h, parts, Wu1, Wu2, b, Wxn):
    BM = 2000

    def body(h_ref, p_ref, w1_ref, w2_ref, b_ref, wx_ref, hn_ref, hw_ref):
        agg = p_ref[0] + p_ref[1]
        u = jnp.dot(h_ref[...], w1_ref[...], preferred_element_type=jnp.float32)
        u = u + jnp.dot(agg, w2_ref[...], preferred_element_type=jnp.float32)
        u = u + b_ref[...]
        hn = h_ref[...] + jnp.maximum(u, 0.0)
        hn_ref[...] = hn
        hw_ref[...] = jnp.dot(hn, wx_ref[...], preferred_element_type=jnp.float32)

    return pl.pallas_call(
        body,
        out_shape=(jax.ShapeDtypeStruct((NN, D), jnp.float32),
                   jax.ShapeDtypeStruct((NN, D), jnp.float32)),
        grid=(NN // BM,),
        in_specs=[pl.BlockSpec((BM, D), lambda i: (i, 0)),
                  pl.BlockSpec((NC, BM, D), lambda i: (0, i, 0)),
                  pl.BlockSpec((D, D), lambda i: (0, 0)),
                  pl.BlockSpec((D, D), lambda i: (0, 0)),
                  pl.BlockSpec((1, D), lambda i: (0, 0)),
                  pl.BlockSpec((D, D), lambda i: (0, 0))],
        out_specs=(pl.BlockSpec((BM, D), lambda i: (i, 0)),
                   pl.BlockSpec((BM, D), lambda i: (i, 0))),
    )(h, parts, Wu1, Wu2, b, Wxn)


def kernel(x, edge_index, batch, edge_attr, length, W_msg, b_msg, W_upd, b_upd):
    del batch
    src = edge_index[0]
    dst = edge_index[1]

    Wx = W_msg[:, :D, :]
    Wc = jnp.concatenate(
        [W_msg[:, D:D + DE + 1, :], b_msg[:, None, :]], axis=1)
    A = jnp.concatenate(
        [edge_attr, length[:, None], jnp.ones((NE, 1), jnp.float32)], axis=1)

    zeros = jnp.zeros((NNP, D), jnp.float32)

    h = x
    hW = _mm(x, Wx[0])
    outs = []
    for i in range(NL):
        Ei = _e_pre(A, Wc[i])
        parts = _edge_kernel()(hW, Ei, src, dst, zeros)
        h, hW = _update(h, parts, W_upd[i, :D, :], W_upd[i, D:, :],
                        b_upd[i][None], Wx[(i + 1) % NL])
        if (i + 1) % 2 == 0:
            outs.append(h)
    return jnp.stack(outs)
